# baseline (speedup 1.0000x reference)
import functools

import jax
import jax.numpy as jnp
from jax import lax
from jax.experimental import pallas as pl
from jax.experimental.pallas import tpu as pltpu
from jax.experimental.pallas import tpu_sc as plsc

VOCAB = 100000
EMBED = 64
BATCH = 16384
MAXLEN = 50

NC = 2
NS = 16
NW = NC * NS

ROWS_PER_W = BATCH // NW
CHUNK_B = 2
CHUNK_TOK = CHUNK_B * MAXLEN
NCHUNK = ROWS_PER_W // CHUNK_B
NBUF = 8


def _body(idx_hbm, table_hbm, out_hbm, idx_v, lists, bufs, out_v, sems):
    wid = lax.axis_index("s") * NC + lax.axis_index("c")
    bcol = wid * ROWS_PER_W

    pltpu.sync_copy(idx_hbm.at[:, pl.ds(bcol, ROWS_PER_W)], idx_v)

    lanes = lax.iota(jnp.int32, 16)

    def build_list(g, b):
        c0 = 2 * g
        c1 = 2 * g + 1
        row = lists[b]
        for k in range(3):
            v = plsc.load_gather(idx_v, [lanes + 16 * k,
                                         jnp.full((16,), c0, jnp.int32)])
            row[pl.ds(16 * k, 16)] = v
        r48 = jnp.where(lanes < 2, lanes + 48, lanes - 2)
        cc = jnp.where(lanes < 2, jnp.full((16,), c0, jnp.int32),
                       jnp.full((16,), c1, jnp.int32))
        row[pl.ds(48, 16)] = plsc.load_gather(idx_v, [r48, cc])
        for k in range(2):
            v = plsc.load_gather(idx_v, [lanes + 14 + 16 * k,
                                         jnp.full((16,), c1, jnp.int32)])
            row[pl.ds(64 + 16 * k, 16)] = v
        mask = lanes < 4
        v = plsc.load_gather(idx_v, [jnp.where(mask, lanes + 46, 0),
                                     jnp.full((16,), c1, jnp.int32)])
        plsc.store_scatter(lists[b], [96 + lanes], v, mask=mask)

    def gather(ls, b):
        pltpu.async_copy(table_hbm.at[lists[ls]], bufs[b], sems[b])

    def wait(ls, b):
        pltpu.make_async_copy(table_hbm.at[lists[ls]], bufs[b],
                              sems[b]).wait()

    def reduce_chunk(g, b):
        buf = bufs[b]
        for r in range(CHUNK_B):
            def lbody(j, accs):
                for u in range(5):
                    row = r * MAXLEN + j * 5 + u
                    accs = tuple(accs[d] + buf[row, pl.ds(d * 16, 16)]
                                 for d in range(4))
                return accs
            accs = lax.fori_loop(
                0, MAXLEN // 5, lbody,
                tuple(jnp.zeros((16,), jnp.float32) for _ in range(4)))
            ocol = jnp.full((16,), CHUNK_B * g + r, jnp.int32)
            for d in range(4):
                plsc.store_scatter(
                    out_v, [lanes + 16 * d, ocol],
                    accs[d] * jnp.float32(1.0 / MAXLEN))

    NLS = 2 * NBUF
    for s in range(NLS):
        build_list(s, s)
    for b in range(NBUF):
        gather(b, b)

    def loop_body(i, _):
        for s in range(NLS):
            g = NLS * i + s
            b = s % NBUF
            wait(s, b)
            reduce_chunk(g, b)
            gather((s + NBUF) % NLS, b)
            build_list(g + NLS, s)
        return 0

    lax.fori_loop(0, NCHUNK // NLS - 1, loop_body, 0)
    for s in range(NLS):
        g = NCHUNK - NLS + s
        b = s % NBUF
        wait(s, b)
        reduce_chunk(g, b)
        if s < NBUF:
            gather((s + NBUF) % NLS, b)

    pltpu.sync_copy(out_v, out_hbm.at[:, pl.ds(bcol, ROWS_PER_W)])


@functools.partial(jax.jit, static_argnames=())
def _run(idx_t, table):
    mesh = plsc.VectorSubcoreMesh(core_axis_name="c", subcore_axis_name="s",
                                  num_cores=NC, num_subcores=NS)
    f = pl.kernel(
        _body,
        out_type=jax.ShapeDtypeStruct((EMBED, BATCH), jnp.float32),
        mesh=mesh,
        scratch_types=[
            pltpu.VMEM((MAXLEN, ROWS_PER_W), jnp.int32),
            [pltpu.VMEM((CHUNK_TOK,), jnp.int32) for _ in range(2 * NBUF)],
            [pltpu.VMEM((CHUNK_TOK, EMBED), jnp.float32)
             for _ in range(NBUF)],
            pltpu.VMEM((EMBED, ROWS_PER_W), jnp.float32),
            [pltpu.SemaphoreType.DMA for _ in range(NBUF)],
        ],
        compiler_params=pltpu.CompilerParams(use_tc_tiling_on_sc=False,
                                             needs_layout_passes=False),
    )
    return f(idx_t, table)


def kernel(indices, table):
    out_t = _run(indices.astype(jnp.int32).T, table)
    return out_t.T


# speedup vs baseline: 1.1313x; 1.1313x over previous
import functools

import jax
import jax.numpy as jnp
from jax import lax
from jax.experimental import pallas as pl
from jax.experimental.pallas import tpu as pltpu
from jax.experimental.pallas import tpu_sc as plsc

VOCAB = 100000
EMBED = 64
BATCH = 16384
MAXLEN = 50

NC = 2
NS = 16
NW = NC * NS

ROWS_PER_W = BATCH // NW
CHUNK_B = 2
CHUNK_TOK = CHUNK_B * MAXLEN
NCHUNK = ROWS_PER_W // CHUNK_B


NBUF = 8


def _body(idx_hbm, table_hbm, out_hbm, idx_v, bufs, out_v, sems):
    wid = lax.axis_index("s") * NC + lax.axis_index("c")
    crow = wid * NCHUNK
    brow = wid * ROWS_PER_W

    pltpu.sync_copy(idx_hbm.at[pl.ds(crow, NCHUNK)], idx_v)

    def gather(g, b):
        pltpu.async_copy(table_hbm.at[idx_v.at[g]], bufs[b], sems[b])

    def wait(g, b):
        pltpu.make_async_copy(table_hbm.at[idx_v.at[g]], bufs[b],
                              sems[b]).wait()

    def reduce_chunk(g, b):
        buf = bufs[b]
        for r in range(CHUNK_B):
            def lbody(j, accs):
                for u in range(5):
                    row = r * MAXLEN + j * 5 + u
                    accs = tuple(accs[d] + buf[row, pl.ds(d * 16, 16)]
                                 for d in range(4))
                return accs
            accs = lax.fori_loop(
                0, MAXLEN // 5, lbody,
                tuple(jnp.zeros((16,), jnp.float32) for _ in range(4)))
            orow = CHUNK_B * g + r
            for d in range(4):
                out_v[orow, pl.ds(d * 16, 16)] = accs[d] * jnp.float32(1.0 / MAXLEN)

    for b in range(NBUF):
        gather(b, b)

    def loop_body(i, _):
        for b in range(NBUF):
            g = NBUF * i + b
            wait(g, b)
            gather_g = g + NBUF
            reduce_chunk(g, b)
            gather(gather_g, b)
        return 0

    lax.fori_loop(0, NCHUNK // NBUF - 1, loop_body, 0)
    for b in range(NBUF):
        g = NCHUNK - NBUF + b
        wait(g, b)
        reduce_chunk(g, b)

    pltpu.sync_copy(out_v, out_hbm.at[pl.ds(brow, ROWS_PER_W)])


@functools.partial(jax.jit, static_argnames=())
def _run(idx2d, table):
    mesh = plsc.VectorSubcoreMesh(core_axis_name="c", subcore_axis_name="s",
                                  num_cores=NC, num_subcores=NS)
    f = pl.kernel(
        _body,
        out_type=jax.ShapeDtypeStruct((BATCH, EMBED), jnp.float32),
        mesh=mesh,
        scratch_types=[
            pltpu.VMEM((NCHUNK, CHUNK_TOK), jnp.int32),
            [pltpu.VMEM((CHUNK_TOK, EMBED), jnp.float32)
             for _ in range(NBUF)],
            pltpu.VMEM((ROWS_PER_W, EMBED), jnp.float32),
            [pltpu.SemaphoreType.DMA for _ in range(NBUF)],
        ],
        compiler_params=pltpu.CompilerParams(use_tc_tiling_on_sc=False),
    )
    return f(idx2d, table)


def kernel(indices, table):
    idx2d = indices.astype(jnp.int32).reshape(BATCH * MAXLEN // CHUNK_TOK,
                                              CHUNK_TOK)
    return _run(idx2d, table)
